# baseline (speedup 1.0000x reference)
import functools

import numpy as np
import jax
import jax.numpy as jnp
from jax import lax
from jax.experimental import pallas as pl
from jax.experimental.pallas import tpu as pltpu
from jax.experimental.pallas import tpu_sc as plsc

_B, _S, _D = 4096, 200, 64
_N = _B * _S
_NC, _NS, _L = 2, 16, 16
_NW = _NC * _NS
_PER_W = _N // _NW
_CHUNK = 100
_NCHUNK = _PER_W // _CHUNK
_NBUF = 2


def _pos_enc8() -> np.ndarray:
    d = np.arange(_D)[np.newaxis, :]
    d = 1.0 / np.power(10000, 2 * (d // 2) / np.float32(_D))
    t = np.arange(_S)[:, np.newaxis] * d
    t = np.concatenate([np.sin(t[:, 0::2]), np.cos(t[:, 1::2])], axis=-1)
    return (t * 8.0).astype(np.float32)


def _make_kernel():
    mesh = plsc.VectorSubcoreMesh(
        core_axis_name="c", subcore_axis_name="s",
        num_cores=_NC, num_subcores=_NS,
    )

    @functools.partial(
        pl.kernel,
        out_type=jax.ShapeDtypeStruct((_N, _D), jnp.float32),
        mesh=mesh,
        scratch_types=[
            pltpu.VMEM((_NCHUNK, _CHUNK), jnp.int32),
            pltpu.VMEM((_S, _D), jnp.float32),
            pltpu.VMEM((_NBUF, _S, _D), jnp.float32),
            pltpu.SemaphoreType.DMA,
            pltpu.SemaphoreType.DMA,
        ],
        compiler_params=pltpu.CompilerParams(use_tc_tiling_on_sc=False),
    )
    def body(y_hbm, pos_hbm, emb_hbm, out_hbm, idx_v, pos_v, buf_v, sem0, sem1):
        sems = (sem0, sem1)
        wid = lax.axis_index("s") * _NC + lax.axis_index("c")
        pltpu.sync_copy(y_hbm.at[pl.ds(wid * _NCHUNK, _NCHUNK)], idx_v)
        pltpu.sync_copy(pos_hbm, pos_v)

        gpb = _S // _CHUNK

        def start_pair(p, b):
            for j in range(gpb):
                pltpu.async_copy(
                    emb_hbm.at[idx_v.at[p * gpb + j]],
                    buf_v.at[b, pl.ds(j * _CHUNK, _CHUNK)],
                    sems[b])

        def wait_pair(p, b):
            for j in range(gpb):
                pltpu.make_async_copy(
                    emb_hbm.at[idx_v.at[p * gpb + j]],
                    buf_v.at[b, pl.ds(j * _CHUNK, _CHUNK)],
                    sems[b]).wait()

        npair = _NCHUNK // gpb
        for b in range(_NBUF):
            start_pair(b, b)

        row0 = wid * _PER_W

        @pl.loop(0, npair, step=_NBUF)
        def _blocks(c):
            for b in range(_NBUF):
                p = c + b
                wait_pair(p, b)

                @pl.loop(0, _S)
                def _rows(r):
                    for k in range(_D // _L):
                        sl = pl.ds(k * _L, _L)
                        buf_v[b, r, sl] = buf_v[b, r, sl] * 8.0 + pos_v[r, sl]

                pltpu.sync_copy(
                    buf_v.at[b], out_hbm.at[pl.ds(row0 + p * _S, _S)])

                nxt = p + _NBUF

                @pl.when(nxt < npair)
                def _():
                    start_pair(nxt, b)

    return body


_EMBED_KERNEL = _make_kernel()
_POS8 = _pos_enc8()


def kernel(y, lens, emb):
    y2 = y.reshape(_NW * _NCHUNK, _CHUNK)
    out = _EMBED_KERNEL(y2, jnp.asarray(_POS8), emb)
    return out.reshape(_B, _S, _D), lens
